# Initial kernel scaffold; baseline (speedup 1.0000x reference)
import jax
import jax.numpy as jnp
from jax.experimental import pallas as pl


def kernel(x, edge_index, edge_weight, W1l, b1l, W1r, g1, be1, W2l, b2l, W2r, g2, be2, Wr1, br1, gr, ber, Wr2, br2):
    raise NotImplementedError("write your pallas kernel here")


# speedup vs baseline: 2.7884x; 2.7884x over previous
import jax
import jax.numpy as jnp
from jax import lax
from jax.experimental import pallas as pl
from jax.experimental.pallas import tpu as pltpu
from jax.experimental.pallas import tpu_sc as plsc

N = 10000
E = 320000
D = 128
NC = 2
NS = 16
NW = NC * NS
C = 128
EP = 327680
EPW = EP // NW
CHUNKS = EPW // C
HCH = CHUNKS // 2
NP = 10240
RPT = NP // NS
ZROWS = 128

_F32 = jnp.float32
_HIGHEST = lax.Precision.HIGHEST


def _matmul_t(a, w):
    return lax.dot_general(a, w, (((1,), (1,)), ((), ())),
                           precision=_HIGHEST, preferred_element_type=_F32)


def _ln(t, g, b):
    mu = jnp.mean(t, axis=1, keepdims=True)
    d = t - mu
    var = jnp.mean(d * d, axis=1, keepdims=True)
    return d * lax.rsqrt(var + 1e-5) * g + b


def _sc_agg_body(y_h, src_h, dst_h, za_h, psum_h,
                 src_v, dst_v, rows0, rows1, acc, sem0, sem1):
    cid = lax.axis_index("c")
    sid = lax.axis_index("s")
    wid = sid * NC + cid

    for k in range(RPT // ZROWS):
        pltpu.sync_copy(za_h, acc.at[pl.ds(sid * RPT + k * ZROWS, ZROWS)])
    plsc.subcore_barrier()

    for p in range(2):
        pltpu.sync_copy(src_h.at[wid, pl.ds(p * HCH, HCH)], src_v)
        pltpu.sync_copy(dst_h.at[wid, pl.ds(p * HCH, HCH)], dst_v)

        pltpu.async_copy(y_h.at[src_v.at[0]], rows0, sem0)

        def mbody(g, carry):
            c0 = 2 * g
            pltpu.make_async_copy(y_h.at[src_v.at[c0]], rows0, sem0).wait()
            pltpu.async_copy(y_h.at[src_v.at[c0 + 1]], rows1, sem1)
            pltpu.sync_copy(rows0, acc.at[dst_v.at[c0]], add=True)
            pltpu.make_async_copy(y_h.at[src_v.at[c0 + 1]], rows1, sem1).wait()

            @pl.when(g < HCH // 2 - 1)
            def _():
                pltpu.async_copy(y_h.at[src_v.at[c0 + 2]], rows0, sem0)

            pltpu.sync_copy(rows1, acc.at[dst_v.at[c0 + 1]], add=True)
            return carry

        lax.fori_loop(0, HCH // 2, mbody, 0)

    plsc.subcore_barrier()

    for k in range(RPT // ZROWS):
        r0 = sid * RPT + k * ZROWS
        pltpu.sync_copy(acc.at[pl.ds(r0, ZROWS)], psum_h.at[cid, pl.ds(r0, ZROWS)])


def _make_sc_agg():
    mesh = plsc.VectorSubcoreMesh(core_axis_name="c", subcore_axis_name="s")
    return pl.kernel(
        _sc_agg_body,
        out_type=[jax.ShapeDtypeStruct((NC, NP, D), _F32)],
        mesh=mesh,
        scratch_types=[
            pltpu.VMEM((HCH, C), jnp.int32),
            pltpu.VMEM((HCH, C), jnp.int32),
            pltpu.VMEM((C, D), _F32),
            pltpu.VMEM((C, D), _F32),
            pltpu.VMEM_SHARED((NP, D), _F32),
            pltpu.SemaphoreType.DMA,
            pltpu.SemaphoreType.DMA,
        ],
    )


_sc_agg_1 = _make_sc_agg()
_sc_agg_2 = _make_sc_agg()


def _sc_cnt_body(dst_h, za_h, ones_h, cnt_h, dst_v, ones_v, cnt_sh):
    cid = lax.axis_index("c")
    sid = lax.axis_index("s")
    wid = sid * NC + cid

    for k in range(RPT // ZROWS):
        pltpu.sync_copy(za_h, cnt_sh.at[pl.ds(sid * RPT + k * ZROWS, ZROWS)])
    pltpu.sync_copy(ones_h, ones_v)
    pltpu.sync_copy(dst_h.at[wid], dst_v)
    plsc.subcore_barrier()

    def cbody(j, carry):
        pltpu.sync_copy(ones_v, cnt_sh.at[dst_v.at[j]], add=True)
        return carry

    lax.fori_loop(0, CHUNKS, cbody, 0)

    plsc.subcore_barrier()
    for k in range(RPT // ZROWS):
        r0 = sid * RPT + k * ZROWS
        pltpu.sync_copy(cnt_sh.at[pl.ds(r0, ZROWS)], cnt_h.at[cid, pl.ds(r0, ZROWS)])


_sc_cnt = pl.kernel(
    _sc_cnt_body,
    out_type=[jax.ShapeDtypeStruct((NC, NP, D), _F32)],
    mesh=plsc.VectorSubcoreMesh(core_axis_name="c", subcore_axis_name="s"),
    scratch_types=[
        pltpu.VMEM((CHUNKS, C), jnp.int32),
        pltpu.VMEM((C, D), _F32),
        pltpu.VMEM_SHARED((NP, D), _F32),
    ],
)


def _tc1_body(x_ref, wl_ref, bl_ref, wr_ref, y_ref, z_ref):
    x = x_ref[...]
    y_ref[...] = _matmul_t(x, wl_ref[...])
    z_ref[...] = _matmul_t(x, wr_ref[...]) + bl_ref[...]


def _tc2_body(p_ref, cnt_ref, z_ref, g_ref, b_ref, wl_ref, bl_ref, wr_ref,
              y_ref, z2_ref):
    inv = 1.0 / jnp.maximum(cnt_ref[0] + cnt_ref[1], 1.0)
    t = (p_ref[0, pl.ds(0, N)] + p_ref[1, pl.ds(0, N)]) * inv + z_ref[...]
    h = jnp.maximum(_ln(t, g_ref[...], b_ref[...]), 0.0)
    y_ref[...] = _matmul_t(h, wl_ref[...])
    z2_ref[...] = _matmul_t(h, wr_ref[...]) + bl_ref[...]


def _tc3_body(p_ref, cnt_ref, z_ref, g_ref, b_ref, wr1_ref, br1_ref, gr_ref,
              ber_ref, wr2_ref, br2_ref, o_ref):
    inv = 1.0 / jnp.maximum(cnt_ref[0] + cnt_ref[1], 1.0)
    t = (p_ref[0, pl.ds(0, N)] + p_ref[1, pl.ds(0, N)]) * inv + z_ref[...]
    h = jnp.maximum(_ln(t, g_ref[...], b_ref[...]), 0.0)
    r = _matmul_t(h, wr1_ref[...]) + br1_ref[...]
    r = jnp.maximum(_ln(r, gr_ref[...], ber_ref[...]), 0.0)
    o_ref[...] = _matmul_t(r, wr2_ref[...]) + br2_ref[...]


_tc1 = pl.pallas_call(
    _tc1_body,
    out_shape=[jax.ShapeDtypeStruct((N, D), _F32),
               jax.ShapeDtypeStruct((N, D), _F32)],
)

_tc2 = pl.pallas_call(
    _tc2_body,
    out_shape=[jax.ShapeDtypeStruct((N, D), _F32),
               jax.ShapeDtypeStruct((N, D), _F32)],
)

_tc3 = pl.pallas_call(
    _tc3_body,
    out_shape=jax.ShapeDtypeStruct((N, 64), _F32),
)


def kernel(x, edge_index, edge_weight, W1l, b1l, W1r, g1, be1,
           W2l, b2l, W2r, g2, be2, Wr1, br1, gr, ber, Wr2, br2):
    del edge_weight
    src = edge_index[0].astype(jnp.int32)
    dst = edge_index[1].astype(jnp.int32)
    pad = jnp.zeros((EP - E,), jnp.int32)
    src2 = jnp.concatenate([src, pad]).reshape(NW, CHUNKS, C)
    dst2 = jnp.concatenate([dst, pad + N]).reshape(NW, CHUNKS, C)
    zeros_a = jnp.zeros((ZROWS, D), _F32)
    ones_rows = jnp.zeros((C, D), _F32).at[:, 0].set(1.0)

    cnt = _sc_cnt(dst2, zeros_a, ones_rows)[0]
    cnt_col = cnt[:, :N, 0:1]
    y1, z1 = _tc1(x, W1l, b1l[None], W1r)
    psum1 = _sc_agg_1(y1, src2, dst2, zeros_a)[0]
    y2, z2 = _tc2(psum1, cnt_col, z1, g1[None], be1[None], W2l, b2l[None], W2r)
    psum2 = _sc_agg_2(y2, src2, dst2, zeros_a)[0]
    out = _tc3(psum2, cnt_col, z2, g2[None], be2[None], Wr1, br1[None],
               gr[None], ber[None], Wr2, br2[None])
    return out
